# baseline (speedup 1.0000x reference)
import functools

import jax
import jax.numpy as jnp
import numpy as np
from jax import lax
from jax.experimental import pallas as pl
from jax.experimental.pallas import tpu as pltpu
from jax.experimental.pallas import tpu_sc as plsc

_LOSS_COEF = 1e-2
_EPS = float(np.finfo(float).eps)

_T = 512
_HB = 1024


def _cv_sq(v):
    n = v.shape[0]
    mu = jnp.mean(v)
    var = jnp.sum((v - mu) ** 2) / (n - 1)
    return var / (mu * mu + 1e-10)


def _routing_kernel(x_ref, wg_ref, loss_ref, g2d_ref, dest_ref, texp_ref,
                    nt_ref, *, maxt):
    x = x_ref[...]
    wg = wg_ref[...]
    B = x.shape[0]
    ne = wg.shape[1]
    logits = lax.dot_general(
        x, wg, (((1,), (0,)), ((), ())), preferred_element_type=jnp.float32
    )
    cols = lax.broadcasted_iota(jnp.int32, logits.shape, 1)
    m1 = jnp.max(logits, axis=1, keepdims=True)
    i1 = jnp.min(jnp.where(logits == m1, cols, ne), axis=1, keepdims=True)
    masked = jnp.where(cols == i1, -jnp.inf, logits)
    m2 = jnp.max(masked, axis=1, keepdims=True)
    i2 = jnp.min(jnp.where(masked == m2, cols, ne), axis=1, keepdims=True)
    e2 = jnp.exp(m2 - m1)
    denom = 1.0 + e2
    g1 = 1.0 / denom
    g2 = e2 / denom

    oh1 = (cols == i1).astype(jnp.float32)
    oh2 = (cols == i2).astype(jnp.float32)
    gates = oh1 * g1 + jnp.where(g2 > 0, oh2 * g2, 0.0)
    importance = jnp.sum(gates, axis=0)
    load = jnp.sum((gates > 0).astype(jnp.float32), axis=0)
    loss_ref[...] = ((_cv_sq(importance) + _cv_sq(load)) * _LOSS_COEF)[
        None, None
    ]

    gcols = lax.broadcasted_iota(jnp.int32, g2d_ref.shape, 1)
    g2d_ref[...] = jnp.where(
        gcols == 0, g1, jnp.where(gcols == 1, g2, 0.0)
    )

    oh = jnp.concatenate([oh1, oh2], axis=0)
    c = oh
    step = 1
    while step < 2 * B:
        c = c + jnp.concatenate(
            [jnp.zeros((step, ne), jnp.float32), c[: 2 * B - step, :]], axis=0
        )
        step *= 2
    counts = c[2 * B - 1 : 2 * B, :]
    cnt_pad = jnp.ceil(counts / _T) * _T
    rl = lax.broadcasted_iota(jnp.int32, (ne, ne), 0)
    cl = lax.broadcasted_iota(jnp.int32, (ne, ne), 1)
    lower = (rl < cl).astype(jnp.float32)
    offs = lax.dot_general(
        cnt_pad, lower, (((1,), (0,)), ((), ())),
        preferred_element_type=jnp.float32,
    )
    ohs = jnp.concatenate([oh1, oh2], axis=0)
    dest = jnp.sum(ohs * (offs + c - 1.0), axis=1, keepdims=True)
    dest_ref[...] = dest.astype(jnp.int32)

    ends = offs + cnt_pad
    jt = lax.broadcasted_iota(jnp.int32, (maxt, 1), 0).astype(jnp.float32) * _T
    texp = jnp.sum((jt >= ends).astype(jnp.float32), axis=1, keepdims=True)
    texp_ref[...] = jnp.minimum(texp, float(ne - 1)).astype(jnp.int32)
    nt_ref[...] = (jnp.sum(cnt_pad) / _T).astype(jnp.int32)[None, None]


def _expert_kernel(texp_ref, nt_ref, xs_ref, W1_ref, b1_ref, W2_ref, b2_ref,
                   out_ref, oe_acc, sem, *, nhb, tt):
    hb = pl.program_id(0)
    j = pl.program_id(1)
    T = xs_ref.shape[0]

    @pl.when(j < nt_ref[0])
    def _():
        rows = pl.ds(j * T, T)
        h = lax.dot_general(
            xs_ref[...].astype(jnp.bfloat16), W1_ref[0],
            (((1,), (0,)), ((), ())),
            preferred_element_type=jnp.float32,
        )
        h = jnp.maximum(h + b1_ref[0], 0.0)
        partial = lax.dot_general(
            h.astype(jnp.bfloat16), W2_ref[0], (((1,), (0,)), ((), ())),
            preferred_element_type=jnp.float32,
        )

        @pl.when(hb == 0)
        def _():
            oe_acc[rows, :] = partial

        @pl.when(hb > 0)
        def _():
            oe_acc[rows, :] += partial

        @pl.when(hb == nhb - 1)
        def _():
            oe_acc[rows, :] = jnp.exp(oe_acc[rows, :] + b2_ref[0])
            copy = pltpu.make_async_copy(
                oe_acc.at[rows, :], out_ref.at[rows, :], sem
            )
            copy.start()
            copy.wait()


def _finalize_kernel(c_ref, g2d_ref, y_ref):
    c0 = c_ref[0]
    c1 = c_ref[1]
    g1 = g2d_ref[:, 0:1]
    g2 = g2d_ref[:, 1:2]
    acc = jnp.where(g1 > 0, g1 * c0, 0.0) + jnp.where(g2 > 0, g2 * c1, 0.0)
    y_ref[...] = jnp.log(jnp.where(acc == 0.0, jnp.float32(_EPS), acc))


def _sc_dispatch(x, dest, buf_rows):
    B, D = x.shape
    info = plsc.get_sparse_core_info()
    NW = info.num_cores * info.num_subcores
    tpw = B // NW

    @functools.partial(
        pl.kernel,
        out_type=jax.ShapeDtypeStruct((buf_rows, D), jnp.float32),
        mesh=plsc.VectorSubcoreMesh(core_axis_name="c", subcore_axis_name="s"),
        scratch_types=[
            pltpu.VMEM((tpw,), jnp.int32),
            pltpu.VMEM((tpw,), jnp.int32),
            pltpu.VMEM((tpw, D), jnp.float32),
            pltpu.SemaphoreType.DMA,
        ],
        compiler_params=pltpu.CompilerParams(needs_layout_passes=False),
    )
    def k(x_hbm, dest_hbm, out_hbm, idx0, idx1, rows_v, sem):
        wid = lax.axis_index("s") * info.num_cores + lax.axis_index("c")
        tb = wid * tpw
        pltpu.sync_copy(x_hbm.at[pl.ds(tb, tpw)], rows_v)
        pltpu.sync_copy(dest_hbm.at[pl.ds(tb, tpw)], idx0)
        pltpu.sync_copy(dest_hbm.at[pl.ds(B + tb, tpw)], idx1)
        pltpu.async_copy(rows_v, out_hbm.at[idx0], sem).wait()
        pltpu.async_copy(rows_v, out_hbm.at[idx1], sem).wait()

    return k(x, dest)


def _sc_gather_rows(table, idx, n_chunks):
    M = idx.shape[0]
    D = table.shape[1]
    info = plsc.get_sparse_core_info()
    NW = info.num_cores * info.num_subcores
    per_w = M // NW
    ch = per_w // n_chunks

    @functools.partial(
        pl.kernel,
        out_type=jax.ShapeDtypeStruct((M, D), jnp.float32),
        mesh=plsc.VectorSubcoreMesh(core_axis_name="c", subcore_axis_name="s"),
        scratch_types=[
            pltpu.VMEM((ch,), jnp.int32),
            pltpu.VMEM((ch, D), jnp.float32),
            pltpu.SemaphoreType.DMA,
        ],
    )
    def k(table_hbm, idx_hbm, out_hbm, idx_v, rows_v, sem):
        wid = lax.axis_index("s") * info.num_cores + lax.axis_index("c")
        base = wid * per_w
        for c in range(n_chunks):
            off = base + c * ch
            pltpu.sync_copy(idx_hbm.at[pl.ds(off, ch)], idx_v)
            pltpu.async_copy(table_hbm.at[idx_v], rows_v, sem).wait()
            pltpu.sync_copy(rows_v, out_hbm.at[pl.ds(off, ch)])

    return k(table, idx)


def kernel(x, w_gate, W1, b1, W2, b2):
    B, D = x.shape
    ne = W1.shape[0]
    H = W1.shape[2]
    O = W2.shape[2]
    hbw = min(_HB, H)
    nhb = H // hbw
    maxt = -(-2 * B // _T) + ne - 1
    maxt += (-maxt) % 8
    buf = maxt * _T

    loss2d, g2d, dest2d, texp2d, nt2d = pl.pallas_call(
        functools.partial(_routing_kernel, maxt=maxt),
        out_shape=(
            jax.ShapeDtypeStruct((1, 1), jnp.float32),
            jax.ShapeDtypeStruct((B, 128), jnp.float32),
            jax.ShapeDtypeStruct((2 * B, 1), jnp.int32),
            jax.ShapeDtypeStruct((maxt, 1), jnp.int32),
            jax.ShapeDtypeStruct((1, 1), jnp.int32),
        ),
    )(x, w_gate)

    dest = dest2d.reshape(2 * B)
    xs = _sc_dispatch(x, dest, buf)

    b1r = b1.reshape(ne, 1, H)
    b2r = b2.reshape(ne, 1, O)
    texp = texp2d.reshape(maxt)
    nt = nt2d.reshape(1)

    contrib = pl.pallas_call(
        functools.partial(_expert_kernel, nhb=nhb, tt=maxt),
        grid_spec=pltpu.PrefetchScalarGridSpec(
            num_scalar_prefetch=2,
            grid=(nhb, maxt),
            in_specs=[
                pl.BlockSpec((_T, D), lambda hb, j, texp, nt: (j, 0)),
                pl.BlockSpec((1, D, hbw), lambda hb, j, texp, nt: (texp[j], 0, hb)),
                pl.BlockSpec((1, 1, hbw), lambda hb, j, texp, nt: (texp[j], 0, hb)),
                pl.BlockSpec((1, hbw, O), lambda hb, j, texp, nt: (texp[j], hb, 0)),
                pl.BlockSpec((1, 1, O), lambda hb, j, texp, nt: (texp[j], 0, 0)),
            ],
            out_specs=pl.BlockSpec(memory_space=pl.ANY),
            scratch_shapes=[
                pltpu.VMEM((buf, O), jnp.float32),
                pltpu.SemaphoreType.DMA,
            ],
        ),
        out_shape=jax.ShapeDtypeStruct((buf, O), jnp.float32),
        compiler_params=pltpu.CompilerParams(
            dimension_semantics=("arbitrary", "arbitrary"),
            vmem_limit_bytes=100 * 1024 * 1024,
        ),
    )(texp, nt, xs, W1.astype(jnp.bfloat16), b1r, W2.astype(jnp.bfloat16),
      b2r)

    crows = _sc_gather_rows(contrib, dest, 2).reshape(2, B, O)

    y = pl.pallas_call(
        _finalize_kernel,
        grid=(B // _T,),
        in_specs=[
            pl.BlockSpec((2, _T, O), lambda t: (0, t, 0)),
            pl.BlockSpec((_T, 128), lambda t: (t, 0)),
        ],
        out_specs=pl.BlockSpec((_T, O), lambda t: (t, 0)),
        out_shape=jax.ShapeDtypeStruct((B, O), jnp.float32),
    )(crows, g2d)

    return y, loss2d[0, 0]


# speedup vs baseline: 1.3958x; 1.3958x over previous
import functools

import jax
import jax.numpy as jnp
import numpy as np
from jax import lax
from jax.experimental import pallas as pl
from jax.experimental.pallas import tpu as pltpu
from jax.experimental.pallas import tpu_sc as plsc

_LOSS_COEF = 1e-2
_EPS = float(np.finfo(float).eps)

_T = 512
_HB = 1024


def _cv_sq(v):
    n = v.shape[0]
    mu = jnp.mean(v)
    var = jnp.sum((v - mu) ** 2) / (n - 1)
    return var / (mu * mu + 1e-10)


def _routing_kernel(x_ref, wg_ref, loss_ref, g2d_ref, dest_ref, texp_ref,
                    nt_ref, *, maxt):
    x = x_ref[...]
    wg = wg_ref[...]
    B = x.shape[0]
    ne = wg.shape[1]
    logits = lax.dot_general(
        x, wg, (((1,), (0,)), ((), ())), preferred_element_type=jnp.float32
    )
    cols = lax.broadcasted_iota(jnp.int32, logits.shape, 1)
    m1 = jnp.max(logits, axis=1, keepdims=True)
    i1 = jnp.min(jnp.where(logits == m1, cols, ne), axis=1, keepdims=True)
    masked = jnp.where(cols == i1, -jnp.inf, logits)
    m2 = jnp.max(masked, axis=1, keepdims=True)
    i2 = jnp.min(jnp.where(masked == m2, cols, ne), axis=1, keepdims=True)
    e2 = jnp.exp(m2 - m1)
    denom = 1.0 + e2
    g1 = 1.0 / denom
    g2 = e2 / denom

    oh1 = (cols == i1).astype(jnp.float32)
    oh2 = (cols == i2).astype(jnp.float32)
    gates = oh1 * g1 + jnp.where(g2 > 0, oh2 * g2, 0.0)
    importance = jnp.sum(gates, axis=0)
    load = jnp.sum((gates > 0).astype(jnp.float32), axis=0)
    loss_ref[...] = ((_cv_sq(importance) + _cv_sq(load)) * _LOSS_COEF)[
        None, None
    ]

    gcols = lax.broadcasted_iota(jnp.int32, g2d_ref.shape, 1)
    g2d_ref[...] = jnp.where(
        gcols == 0, g1, jnp.where(gcols == 1, g2, 0.0)
    )

    oh = jnp.concatenate([oh1, oh2], axis=0)
    c = oh
    step = 1
    while step < 2 * B:
        c = c + jnp.concatenate(
            [jnp.zeros((step, ne), jnp.float32), c[: 2 * B - step, :]], axis=0
        )
        step *= 2
    counts = c[2 * B - 1 : 2 * B, :]
    cnt_pad = jnp.ceil(counts / _T) * _T
    rl = lax.broadcasted_iota(jnp.int32, (ne, ne), 0)
    cl = lax.broadcasted_iota(jnp.int32, (ne, ne), 1)
    lower = (rl < cl).astype(jnp.float32)
    offs = lax.dot_general(
        cnt_pad, lower, (((1,), (0,)), ((), ())),
        preferred_element_type=jnp.float32,
    )
    ohs = jnp.concatenate([oh1, oh2], axis=0)
    dest = jnp.sum(ohs * (offs + c - 1.0), axis=1, keepdims=True)
    dest_ref[...] = dest.astype(jnp.int32)

    ends = offs + cnt_pad
    jt = lax.broadcasted_iota(jnp.int32, (maxt, 1), 0).astype(jnp.float32) * _T
    texp = jnp.sum((jt >= ends).astype(jnp.float32), axis=1, keepdims=True)
    texp_ref[...] = jnp.minimum(texp, float(ne - 1)).astype(jnp.int32)
    nt_ref[...] = (jnp.sum(cnt_pad) / _T).astype(jnp.int32)[None, None]


def _expert_kernel(texp_ref, nt_ref, xs_ref, W1_ref, b1_ref, W2_ref, b2_ref,
                   out_ref, oe_acc, sem, *, nhb, tt):
    hb = pl.program_id(0)
    j = pl.program_id(1)
    T = xs_ref.shape[0]

    @pl.when(j < nt_ref[0])
    def _():
        rows = pl.ds(j * T, T)
        h = lax.dot_general(
            xs_ref[...], W1_ref[0], (((1,), (0,)), ((), ())),
            preferred_element_type=jnp.float32,
        )
        h = jnp.maximum(h + b1_ref[0], 0.0)
        partial = lax.dot_general(
            h, W2_ref[0], (((1,), (0,)), ((), ())),
            preferred_element_type=jnp.float32,
        )

        @pl.when(hb == 0)
        def _():
            oe_acc[rows, :] = partial

        @pl.when(hb > 0)
        def _():
            oe_acc[rows, :] += partial

        @pl.when(hb == nhb - 1)
        def _():
            oe_acc[rows, :] = jnp.exp(oe_acc[rows, :] + b2_ref[0])

            @pl.when(j > 0)
            def _():
                prows = pl.ds((j - 1) * T, T)
                pltpu.make_async_copy(
                    oe_acc.at[prows, :], out_ref.at[prows, :], sem
                ).wait()

            pltpu.make_async_copy(
                oe_acc.at[rows, :], out_ref.at[rows, :], sem
            ).start()

    @pl.when(jnp.logical_and(hb == nhb - 1, j == tt - 1))
    def _():
        lrows = pl.ds((nt_ref[0] - 1) * T, T)
        pltpu.make_async_copy(
            oe_acc.at[lrows, :], out_ref.at[lrows, :], sem
        ).wait()


def _finalize_kernel(c_ref, g2d_ref, y_ref):
    c0 = c_ref[0]
    c1 = c_ref[1]
    g1 = g2d_ref[:, 0:1]
    g2 = g2d_ref[:, 1:2]
    acc = jnp.where(g1 > 0, g1 * c0, 0.0) + jnp.where(g2 > 0, g2 * c1, 0.0)
    y_ref[...] = jnp.log(jnp.where(acc == 0.0, jnp.float32(_EPS), acc))


def _sc_dispatch(x, dest, buf_rows):
    B, D = x.shape
    info = plsc.get_sparse_core_info()
    NW = info.num_cores * info.num_subcores
    tpw = B // NW

    @functools.partial(
        pl.kernel,
        out_type=jax.ShapeDtypeStruct((buf_rows, D), jnp.float32),
        mesh=plsc.VectorSubcoreMesh(core_axis_name="c", subcore_axis_name="s"),
        scratch_types=[
            pltpu.VMEM((tpw,), jnp.int32),
            pltpu.VMEM((tpw,), jnp.int32),
            pltpu.VMEM((tpw, D), jnp.float32),
            pltpu.SemaphoreType.DMA,
        ],
        compiler_params=pltpu.CompilerParams(needs_layout_passes=False),
    )
    def k(x_hbm, dest_hbm, out_hbm, idx0, idx1, rows_v, sem):
        wid = lax.axis_index("s") * info.num_cores + lax.axis_index("c")
        tb = wid * tpw
        pltpu.sync_copy(x_hbm.at[pl.ds(tb, tpw)], rows_v)
        pltpu.sync_copy(dest_hbm.at[pl.ds(tb, tpw)], idx0)
        pltpu.sync_copy(dest_hbm.at[pl.ds(B + tb, tpw)], idx1)
        pltpu.async_copy(rows_v, out_hbm.at[idx0], sem).wait()
        pltpu.async_copy(rows_v, out_hbm.at[idx1], sem).wait()

    return k(x, dest)


def _sc_gather_rows(table, idx, n_chunks):
    M = idx.shape[0]
    D = table.shape[1]
    info = plsc.get_sparse_core_info()
    NW = info.num_cores * info.num_subcores
    per_w = M // NW
    ch = per_w // n_chunks

    @functools.partial(
        pl.kernel,
        out_type=jax.ShapeDtypeStruct((M, D), jnp.float32),
        mesh=plsc.VectorSubcoreMesh(core_axis_name="c", subcore_axis_name="s"),
        scratch_types=[
            pltpu.VMEM((ch,), jnp.int32),
            pltpu.VMEM((ch, D), jnp.float32),
            pltpu.SemaphoreType.DMA,
        ],
    )
    def k(table_hbm, idx_hbm, out_hbm, idx_v, rows_v, sem):
        wid = lax.axis_index("s") * info.num_cores + lax.axis_index("c")
        base = wid * per_w
        for c in range(n_chunks):
            off = base + c * ch
            pltpu.sync_copy(idx_hbm.at[pl.ds(off, ch)], idx_v)
            pltpu.async_copy(table_hbm.at[idx_v], rows_v, sem).wait()
            pltpu.sync_copy(rows_v, out_hbm.at[pl.ds(off, ch)])

    return k(table, idx)


def kernel(x, w_gate, W1, b1, W2, b2):
    B, D = x.shape
    ne = W1.shape[0]
    H = W1.shape[2]
    O = W2.shape[2]
    hbw = min(_HB, H)
    nhb = H // hbw
    maxt = -(-2 * B // _T) + ne - 1
    maxt += (-maxt) % 8
    buf = maxt * _T

    loss2d, g2d, dest2d, texp2d, nt2d = pl.pallas_call(
        functools.partial(_routing_kernel, maxt=maxt),
        out_shape=(
            jax.ShapeDtypeStruct((1, 1), jnp.float32),
            jax.ShapeDtypeStruct((B, 128), jnp.float32),
            jax.ShapeDtypeStruct((2 * B, 1), jnp.int32),
            jax.ShapeDtypeStruct((maxt, 1), jnp.int32),
            jax.ShapeDtypeStruct((1, 1), jnp.int32),
        ),
    )(x, w_gate)

    dest = dest2d.reshape(2 * B)
    xs = _sc_dispatch(x, dest, buf)

    b1r = b1.reshape(ne, 1, H)
    b2r = b2.reshape(ne, 1, O)
    texp = texp2d.reshape(maxt)
    nt = nt2d.reshape(1)

    contrib = pl.pallas_call(
        functools.partial(_expert_kernel, nhb=nhb, tt=maxt),
        grid_spec=pltpu.PrefetchScalarGridSpec(
            num_scalar_prefetch=2,
            grid=(nhb, maxt),
            in_specs=[
                pl.BlockSpec((_T, D), lambda hb, j, texp, nt: (j, 0)),
                pl.BlockSpec((1, D, hbw), lambda hb, j, texp, nt: (texp[j], 0, hb)),
                pl.BlockSpec((1, 1, hbw), lambda hb, j, texp, nt: (texp[j], 0, hb)),
                pl.BlockSpec((1, hbw, O), lambda hb, j, texp, nt: (texp[j], hb, 0)),
                pl.BlockSpec((1, 1, O), lambda hb, j, texp, nt: (texp[j], 0, 0)),
            ],
            out_specs=pl.BlockSpec(memory_space=pl.ANY),
            scratch_shapes=[
                pltpu.VMEM((buf, O), jnp.float32),
                pltpu.SemaphoreType.DMA,
            ],
        ),
        out_shape=jax.ShapeDtypeStruct((buf, O), jnp.float32),
        compiler_params=pltpu.CompilerParams(
            dimension_semantics=("arbitrary", "arbitrary"),
            vmem_limit_bytes=100 * 1024 * 1024,
        ),
    )(texp, nt, xs, W1, b1r, W2, b2r)

    crows = _sc_gather_rows(contrib, dest, 2).reshape(2, B, O)

    y = pl.pallas_call(
        _finalize_kernel,
        grid=(B // _T,),
        in_specs=[
            pl.BlockSpec((2, _T, O), lambda t: (0, t, 0)),
            pl.BlockSpec((_T, 128), lambda t: (t, 0)),
        ],
        out_specs=pl.BlockSpec((_T, O), lambda t: (t, 0)),
        out_shape=jax.ShapeDtypeStruct((B, O), jnp.float32),
    )(crows, g2d)

    return y, loss2d[0, 0]


# speedup vs baseline: 1.3990x; 1.0023x over previous
import functools

import jax
import jax.numpy as jnp
import numpy as np
from jax import lax
from jax.experimental import pallas as pl
from jax.experimental.pallas import tpu as pltpu
from jax.experimental.pallas import tpu_sc as plsc

_LOSS_COEF = 1e-2
_EPS = float(np.finfo(float).eps)

_T = 512
_HB = 1024


def _cv_sq(v):
    n = v.shape[0]
    mu = jnp.mean(v)
    var = jnp.sum((v - mu) ** 2) / (n - 1)
    return var / (mu * mu + 1e-10)


def _routing_kernel(x_ref, wg_ref, loss_ref, g2d_ref, dest_ref, texp_ref,
                    nt_ref, *, maxt):
    x = x_ref[...]
    wg = wg_ref[...]
    B = x.shape[0]
    ne = wg.shape[1]
    logits = lax.dot_general(
        x, wg, (((1,), (0,)), ((), ())), preferred_element_type=jnp.float32
    )
    cols = lax.broadcasted_iota(jnp.int32, logits.shape, 1)
    m1 = jnp.max(logits, axis=1, keepdims=True)
    i1 = jnp.min(jnp.where(logits == m1, cols, ne), axis=1, keepdims=True)
    masked = jnp.where(cols == i1, -jnp.inf, logits)
    m2 = jnp.max(masked, axis=1, keepdims=True)
    i2 = jnp.min(jnp.where(masked == m2, cols, ne), axis=1, keepdims=True)
    e2 = jnp.exp(m2 - m1)
    denom = 1.0 + e2
    g1 = 1.0 / denom
    g2 = e2 / denom

    oh1 = (cols == i1).astype(jnp.float32)
    oh2 = (cols == i2).astype(jnp.float32)
    gates = oh1 * g1 + jnp.where(g2 > 0, oh2 * g2, 0.0)
    importance = jnp.sum(gates, axis=0)
    load = jnp.sum((gates > 0).astype(jnp.float32), axis=0)
    loss_ref[...] = ((_cv_sq(importance) + _cv_sq(load)) * _LOSS_COEF)[
        None, None
    ]

    gcols = lax.broadcasted_iota(jnp.int32, g2d_ref.shape, 1)
    g2d_ref[...] = jnp.where(
        gcols == 0, g1, jnp.where(gcols == 1, g2, 0.0)
    )

    oh = jnp.concatenate([oh1, oh2], axis=0)
    c = oh
    step = 1
    while step < 2 * B:
        c = c + jnp.concatenate(
            [jnp.zeros((step, ne), jnp.float32), c[: 2 * B - step, :]], axis=0
        )
        step *= 2
    counts = c[2 * B - 1 : 2 * B, :]
    cnt_pad = jnp.ceil(counts / _T) * _T
    rl = lax.broadcasted_iota(jnp.int32, (ne, ne), 0)
    cl = lax.broadcasted_iota(jnp.int32, (ne, ne), 1)
    lower = (rl < cl).astype(jnp.float32)
    offs = lax.dot_general(
        cnt_pad, lower, (((1,), (0,)), ((), ())),
        preferred_element_type=jnp.float32,
    )
    ohs = jnp.concatenate([oh1, oh2], axis=0)
    dest = jnp.sum(ohs * (offs + c - 1.0), axis=1, keepdims=True)
    dest_ref[...] = dest.astype(jnp.int32)

    ends = offs + cnt_pad
    jt = lax.broadcasted_iota(jnp.int32, (maxt, 1), 0).astype(jnp.float32) * _T
    texp = jnp.sum((jt >= ends).astype(jnp.float32), axis=1, keepdims=True)
    texp_ref[...] = jnp.minimum(texp, float(ne - 1)).astype(jnp.int32)
    nt_ref[...] = (jnp.sum(cnt_pad) / _T).astype(jnp.int32)[None, None]


def _expert_kernel(texp_ref, nt_ref, xs_ref, W1_ref, b1_ref, W2_ref, b2_ref,
                   out_ref, oe_acc, sem, *, nhb, tt):
    hb = pl.program_id(0)
    j = pl.program_id(1)
    T = xs_ref.shape[0]

    @pl.when(j < nt_ref[0])
    def _():
        rows = pl.ds(j * T, T)
        h = lax.dot_general(
            xs_ref[...], W1_ref[0], (((1,), (0,)), ((), ())),
            preferred_element_type=jnp.float32,
        )
        h = jnp.maximum(h + b1_ref[0], 0.0)
        partial = lax.dot_general(
            h, W2_ref[0], (((1,), (0,)), ((), ())),
            preferred_element_type=jnp.float32,
        )

        @pl.when(hb == 0)
        def _():
            oe_acc[rows, :] = partial

        @pl.when(hb > 0)
        def _():
            oe_acc[rows, :] += partial

        @pl.when(hb == nhb - 1)
        def _():
            oe_acc[rows, :] = jnp.exp(oe_acc[rows, :] + b2_ref[0])

            @pl.when(j > 0)
            def _():
                prows = pl.ds((j - 1) * T, T)
                pltpu.make_async_copy(
                    oe_acc.at[prows, :], out_ref.at[prows, :], sem
                ).wait()

            pltpu.make_async_copy(
                oe_acc.at[rows, :], out_ref.at[rows, :], sem
            ).start()

    @pl.when(jnp.logical_and(hb == nhb - 1, j == tt - 1))
    def _():
        lrows = pl.ds((nt_ref[0] - 1) * T, T)
        pltpu.make_async_copy(
            oe_acc.at[lrows, :], out_ref.at[lrows, :], sem
        ).wait()


def _finalize_kernel(c_ref, g2d_ref, y_ref):
    c0 = c_ref[0]
    c1 = c_ref[1]
    g1 = g2d_ref[:, 0:1]
    g2 = g2d_ref[:, 1:2]
    acc = jnp.where(g1 > 0, g1 * c0, 0.0) + jnp.where(g2 > 0, g2 * c1, 0.0)
    y_ref[...] = jnp.log(jnp.where(acc == 0.0, jnp.float32(_EPS), acc))


def _sc_dispatch(x, dest, buf_rows):
    B, D = x.shape
    info = plsc.get_sparse_core_info()
    NW = info.num_cores * info.num_subcores
    tpw = B // NW

    @functools.partial(
        pl.kernel,
        out_type=jax.ShapeDtypeStruct((buf_rows, D), jnp.float32),
        mesh=plsc.VectorSubcoreMesh(core_axis_name="c", subcore_axis_name="s"),
        scratch_types=[
            pltpu.VMEM((tpw,), jnp.int32),
            pltpu.VMEM((tpw,), jnp.int32),
            pltpu.VMEM((tpw, D), jnp.float32),
            pltpu.SemaphoreType.DMA,
        ],
        compiler_params=pltpu.CompilerParams(needs_layout_passes=False),
    )
    def k(x_hbm, dest_hbm, out_hbm, idx0, idx1, rows_v, sem):
        wid = lax.axis_index("s") * info.num_cores + lax.axis_index("c")
        tb = wid * tpw
        pltpu.sync_copy(x_hbm.at[pl.ds(tb, tpw)], rows_v)
        pltpu.sync_copy(dest_hbm.at[pl.ds(tb, tpw)], idx0)
        pltpu.sync_copy(dest_hbm.at[pl.ds(B + tb, tpw)], idx1)
        c0 = pltpu.async_copy(rows_v, out_hbm.at[idx0], sem)
        c1 = pltpu.async_copy(rows_v, out_hbm.at[idx1], sem)
        c0.wait()
        c1.wait()

    return k(x, dest)


def _sc_gather_rows(table, idx, n_chunks):
    M = idx.shape[0]
    D = table.shape[1]
    info = plsc.get_sparse_core_info()
    NW = info.num_cores * info.num_subcores
    per_w = M // NW
    ch = per_w // n_chunks

    @functools.partial(
        pl.kernel,
        out_type=jax.ShapeDtypeStruct((M, D), jnp.float32),
        mesh=plsc.VectorSubcoreMesh(core_axis_name="c", subcore_axis_name="s"),
        scratch_types=[
            pltpu.VMEM((ch,), jnp.int32),
            pltpu.VMEM((ch, D), jnp.float32),
            pltpu.SemaphoreType.DMA,
        ],
    )
    def k(table_hbm, idx_hbm, out_hbm, idx_v, rows_v, sem):
        wid = lax.axis_index("s") * info.num_cores + lax.axis_index("c")
        base = wid * per_w
        for c in range(n_chunks):
            off = base + c * ch
            pltpu.sync_copy(idx_hbm.at[pl.ds(off, ch)], idx_v)
            pltpu.async_copy(table_hbm.at[idx_v], rows_v, sem).wait()
            pltpu.sync_copy(rows_v, out_hbm.at[pl.ds(off, ch)])

    return k(table, idx)


def kernel(x, w_gate, W1, b1, W2, b2):
    B, D = x.shape
    ne = W1.shape[0]
    H = W1.shape[2]
    O = W2.shape[2]
    hbw = min(_HB, H)
    nhb = H // hbw
    maxt = -(-2 * B // _T) + ne - 1
    maxt += (-maxt) % 8
    buf = maxt * _T

    loss2d, g2d, dest2d, texp2d, nt2d = pl.pallas_call(
        functools.partial(_routing_kernel, maxt=maxt),
        out_shape=(
            jax.ShapeDtypeStruct((1, 1), jnp.float32),
            jax.ShapeDtypeStruct((B, 128), jnp.float32),
            jax.ShapeDtypeStruct((2 * B, 1), jnp.int32),
            jax.ShapeDtypeStruct((maxt, 1), jnp.int32),
            jax.ShapeDtypeStruct((1, 1), jnp.int32),
        ),
    )(x, w_gate)

    dest = dest2d.reshape(2 * B)
    xs = _sc_dispatch(x, dest, buf)

    b1r = b1.reshape(ne, 1, H)
    b2r = b2.reshape(ne, 1, O)
    texp = texp2d.reshape(maxt)
    nt = nt2d.reshape(1)

    contrib = pl.pallas_call(
        functools.partial(_expert_kernel, nhb=nhb, tt=maxt),
        grid_spec=pltpu.PrefetchScalarGridSpec(
            num_scalar_prefetch=2,
            grid=(nhb, maxt),
            in_specs=[
                pl.BlockSpec((_T, D), lambda hb, j, texp, nt: (j, 0)),
                pl.BlockSpec((1, D, hbw), lambda hb, j, texp, nt: (texp[j], 0, hb)),
                pl.BlockSpec((1, 1, hbw), lambda hb, j, texp, nt: (texp[j], 0, hb)),
                pl.BlockSpec((1, hbw, O), lambda hb, j, texp, nt: (texp[j], hb, 0)),
                pl.BlockSpec((1, 1, O), lambda hb, j, texp, nt: (texp[j], 0, 0)),
            ],
            out_specs=pl.BlockSpec(memory_space=pl.ANY),
            scratch_shapes=[
                pltpu.VMEM((buf, O), jnp.float32),
                pltpu.SemaphoreType.DMA,
            ],
        ),
        out_shape=jax.ShapeDtypeStruct((buf, O), jnp.float32),
        compiler_params=pltpu.CompilerParams(
            dimension_semantics=("arbitrary", "arbitrary"),
            vmem_limit_bytes=100 * 1024 * 1024,
        ),
    )(texp, nt, xs, W1, b1r, W2, b2r)

    crows = _sc_gather_rows(contrib, dest, 2).reshape(2, B, O)

    y = pl.pallas_call(
        _finalize_kernel,
        grid=(B // _T,),
        in_specs=[
            pl.BlockSpec((2, _T, O), lambda t: (0, t, 0)),
            pl.BlockSpec((_T, 128), lambda t: (t, 0)),
        ],
        out_specs=pl.BlockSpec((_T, O), lambda t: (t, 0)),
        out_shape=jax.ShapeDtypeStruct((B, O), jnp.float32),
    )(crows, g2d)

    return y, loss2d[0, 0]


# speedup vs baseline: 1.3994x; 1.0003x over previous
import functools

import jax
import jax.numpy as jnp
import numpy as np
from jax import lax
from jax.experimental import pallas as pl
from jax.experimental.pallas import tpu as pltpu
from jax.experimental.pallas import tpu_sc as plsc

_LOSS_COEF = 1e-2
_EPS = float(np.finfo(float).eps)

_T = 512
_HB = 1024


def _cv_sq(v):
    n = v.shape[0]
    mu = jnp.mean(v)
    var = jnp.sum((v - mu) ** 2) / (n - 1)
    return var / (mu * mu + 1e-10)


def _routing_kernel(x_ref, wg_ref, loss_ref, g2d_ref, dest_ref, texp_ref,
                    nt_ref, *, maxt):
    x = x_ref[...]
    wg = wg_ref[...]
    B = x.shape[0]
    ne = wg.shape[1]
    logits = lax.dot_general(
        x, wg, (((1,), (0,)), ((), ())), preferred_element_type=jnp.float32
    )
    cols = lax.broadcasted_iota(jnp.int32, logits.shape, 1)
    m1 = jnp.max(logits, axis=1, keepdims=True)
    i1 = jnp.min(jnp.where(logits == m1, cols, ne), axis=1, keepdims=True)
    masked = jnp.where(cols == i1, -jnp.inf, logits)
    m2 = jnp.max(masked, axis=1, keepdims=True)
    i2 = jnp.min(jnp.where(masked == m2, cols, ne), axis=1, keepdims=True)
    e2 = jnp.exp(m2 - m1)
    denom = 1.0 + e2
    g1 = 1.0 / denom
    g2 = e2 / denom

    oh1 = (cols == i1).astype(jnp.float32)
    oh2 = (cols == i2).astype(jnp.float32)
    gates = oh1 * g1 + jnp.where(g2 > 0, oh2 * g2, 0.0)
    importance = jnp.sum(gates, axis=0)
    load = jnp.sum((gates > 0).astype(jnp.float32), axis=0)
    loss_ref[...] = ((_cv_sq(importance) + _cv_sq(load)) * _LOSS_COEF)[
        None, None
    ]

    gcols = lax.broadcasted_iota(jnp.int32, g2d_ref.shape, 1)
    g2d_ref[...] = jnp.where(
        gcols == 0, g1, jnp.where(gcols == 1, g2, 0.0)
    )

    oh = jnp.concatenate([oh1, oh2], axis=0)
    c = oh
    step = 1
    while step < 2 * B:
        c = c + jnp.concatenate(
            [jnp.zeros((step, ne), jnp.float32), c[: 2 * B - step, :]], axis=0
        )
        step *= 2
    counts = c[2 * B - 1 : 2 * B, :]
    cnt_pad = jnp.ceil(counts / _T) * _T
    rl = lax.broadcasted_iota(jnp.int32, (ne, ne), 0)
    cl = lax.broadcasted_iota(jnp.int32, (ne, ne), 1)
    lower = (rl < cl).astype(jnp.float32)
    offs = lax.dot_general(
        cnt_pad, lower, (((1,), (0,)), ((), ())),
        preferred_element_type=jnp.float32,
    )
    ohs = jnp.concatenate([oh1, oh2], axis=0)
    dest = jnp.sum(ohs * (offs + c - 1.0), axis=1, keepdims=True)
    dest_ref[...] = dest.astype(jnp.int32)

    ends = offs + cnt_pad
    jt = lax.broadcasted_iota(jnp.int32, (maxt, 1), 0).astype(jnp.float32) * _T
    texp = jnp.sum((jt >= ends).astype(jnp.float32), axis=1, keepdims=True)
    texp_ref[...] = jnp.minimum(texp, float(ne - 1)).astype(jnp.int32)
    nt_ref[...] = (jnp.sum(cnt_pad) / _T).astype(jnp.int32)[None, None]


def _expert_kernel(texp_ref, nt_ref, xs_ref, W1_ref, b1_ref, W2_ref, b2_ref,
                   out_ref, oe_acc, sem, *, nhb, tt):
    hb = pl.program_id(0)
    j = pl.program_id(1)
    T = xs_ref.shape[0]

    @pl.when(j < nt_ref[0])
    def _():
        rows = pl.ds(j * T, T)
        h = lax.dot_general(
            xs_ref[...], W1_ref[0], (((1,), (0,)), ((), ())),
            preferred_element_type=jnp.float32,
        )
        h = jnp.maximum(h + b1_ref[0], 0.0)
        partial = lax.dot_general(
            h, W2_ref[0], (((1,), (0,)), ((), ())),
            preferred_element_type=jnp.float32,
        )

        @pl.when(hb == 0)
        def _():
            oe_acc[rows, :] = partial

        @pl.when(hb > 0)
        def _():
            oe_acc[rows, :] += partial

        @pl.when(hb == nhb - 1)
        def _():
            oe_acc[rows, :] = jnp.exp(oe_acc[rows, :] + b2_ref[0])

            @pl.when(j > 0)
            def _():
                prows = pl.ds((j - 1) * T, T)
                pltpu.make_async_copy(
                    oe_acc.at[prows, :], out_ref.at[prows, :], sem
                ).wait()

            pltpu.make_async_copy(
                oe_acc.at[rows, :], out_ref.at[rows, :], sem
            ).start()

    @pl.when(jnp.logical_and(hb == nhb - 1, j == tt - 1))
    def _():
        lrows = pl.ds((nt_ref[0] - 1) * T, T)
        pltpu.make_async_copy(
            oe_acc.at[lrows, :], out_ref.at[lrows, :], sem
        ).wait()


def _finalize_kernel(c_ref, g2d_ref, y_ref):
    c0 = c_ref[0]
    c1 = c_ref[1]
    g1 = g2d_ref[:, 0:1]
    g2 = g2d_ref[:, 1:2]
    acc = jnp.where(g1 > 0, g1 * c0, 0.0) + jnp.where(g2 > 0, g2 * c1, 0.0)
    y_ref[...] = jnp.log(jnp.where(acc == 0.0, jnp.float32(_EPS), acc))


def _sc_dispatch(x, dest, buf_rows):
    B, D = x.shape
    info = plsc.get_sparse_core_info()
    NW = info.num_cores * info.num_subcores
    tpw = B // NW

    @functools.partial(
        pl.kernel,
        out_type=jax.ShapeDtypeStruct((buf_rows, D), jnp.float32),
        mesh=plsc.VectorSubcoreMesh(core_axis_name="c", subcore_axis_name="s"),
        scratch_types=[
            pltpu.VMEM((tpw,), jnp.int32),
            pltpu.VMEM((tpw,), jnp.int32),
            pltpu.VMEM((tpw, D), jnp.float32),
            pltpu.SemaphoreType.DMA,
        ],
        compiler_params=pltpu.CompilerParams(needs_layout_passes=False),
    )
    def k(x_hbm, dest_hbm, out_hbm, idx0, idx1, rows_v, sem):
        wid = lax.axis_index("s") * info.num_cores + lax.axis_index("c")
        tb = wid * tpw
        pltpu.sync_copy(x_hbm.at[pl.ds(tb, tpw)], rows_v)
        pltpu.sync_copy(dest_hbm.at[pl.ds(tb, tpw)], idx0)
        pltpu.sync_copy(dest_hbm.at[pl.ds(B + tb, tpw)], idx1)
        c0 = pltpu.async_copy(rows_v, out_hbm.at[idx0], sem)
        c1 = pltpu.async_copy(rows_v, out_hbm.at[idx1], sem)
        c0.wait()
        c1.wait()

    return k(x, dest)


def _sc_gather_rows(table, idx, n_chunks):
    M = idx.shape[0]
    D = table.shape[1]
    info = plsc.get_sparse_core_info()
    NW = info.num_cores * info.num_subcores
    per_w = M // NW
    ch = per_w // n_chunks

    @functools.partial(
        pl.kernel,
        out_type=jax.ShapeDtypeStruct((M, D), jnp.float32),
        mesh=plsc.VectorSubcoreMesh(core_axis_name="c", subcore_axis_name="s"),
        scratch_types=[
            pltpu.VMEM((2, ch), jnp.int32),
            pltpu.VMEM((2, ch, D), jnp.float32),
            pltpu.SemaphoreType.DMA,
        ],
    )
    def k(table_hbm, idx_hbm, out_hbm, idx_v, rows_v, sem):
        wid = lax.axis_index("s") * info.num_cores + lax.axis_index("c")
        base = wid * per_w

        def start(c):
            b = c % 2
            off = base + c * ch
            pltpu.sync_copy(idx_hbm.at[pl.ds(off, ch)], idx_v.at[b])
            return pltpu.async_copy(
                table_hbm.at[idx_v.at[b]], rows_v.at[b], sem
            )

        handles = [None] * n_chunks
        for c in range(min(2, n_chunks)):
            handles[c] = start(c)
        for c in range(n_chunks):
            handles[c].wait()
            off = base + c * ch
            pltpu.sync_copy(rows_v.at[c % 2], out_hbm.at[pl.ds(off, ch)])
            if c + 2 < n_chunks:
                handles[c + 2] = start(c + 2)

    return k(table, idx)


def kernel(x, w_gate, W1, b1, W2, b2):
    B, D = x.shape
    ne = W1.shape[0]
    H = W1.shape[2]
    O = W2.shape[2]
    hbw = min(_HB, H)
    nhb = H // hbw
    maxt = -(-2 * B // _T) + ne - 1
    maxt += (-maxt) % 8
    buf = maxt * _T

    loss2d, g2d, dest2d, texp2d, nt2d = pl.pallas_call(
        functools.partial(_routing_kernel, maxt=maxt),
        out_shape=(
            jax.ShapeDtypeStruct((1, 1), jnp.float32),
            jax.ShapeDtypeStruct((B, 128), jnp.float32),
            jax.ShapeDtypeStruct((2 * B, 1), jnp.int32),
            jax.ShapeDtypeStruct((maxt, 1), jnp.int32),
            jax.ShapeDtypeStruct((1, 1), jnp.int32),
        ),
    )(x, w_gate)

    dest = dest2d.reshape(2 * B)
    xs = _sc_dispatch(x, dest, buf)

    b1r = b1.reshape(ne, 1, H)
    b2r = b2.reshape(ne, 1, O)
    texp = texp2d.reshape(maxt)
    nt = nt2d.reshape(1)

    contrib = pl.pallas_call(
        functools.partial(_expert_kernel, nhb=nhb, tt=maxt),
        grid_spec=pltpu.PrefetchScalarGridSpec(
            num_scalar_prefetch=2,
            grid=(nhb, maxt),
            in_specs=[
                pl.BlockSpec((_T, D), lambda hb, j, texp, nt: (j, 0)),
                pl.BlockSpec((1, D, hbw), lambda hb, j, texp, nt: (texp[j], 0, hb)),
                pl.BlockSpec((1, 1, hbw), lambda hb, j, texp, nt: (texp[j], 0, hb)),
                pl.BlockSpec((1, hbw, O), lambda hb, j, texp, nt: (texp[j], hb, 0)),
                pl.BlockSpec((1, 1, O), lambda hb, j, texp, nt: (texp[j], 0, 0)),
            ],
            out_specs=pl.BlockSpec(memory_space=pl.ANY),
            scratch_shapes=[
                pltpu.VMEM((buf, O), jnp.float32),
                pltpu.SemaphoreType.DMA,
            ],
        ),
        out_shape=jax.ShapeDtypeStruct((buf, O), jnp.float32),
        compiler_params=pltpu.CompilerParams(
            dimension_semantics=("arbitrary", "arbitrary"),
            vmem_limit_bytes=100 * 1024 * 1024,
        ),
    )(texp, nt, xs, W1, b1r, W2, b2r)

    crows = _sc_gather_rows(contrib, dest, 4).reshape(2, B, O)

    y = pl.pallas_call(
        _finalize_kernel,
        grid=(B // _T,),
        in_specs=[
            pl.BlockSpec((2, _T, O), lambda t: (0, t, 0)),
            pl.BlockSpec((_T, 128), lambda t: (t, 0)),
        ],
        out_specs=pl.BlockSpec((_T, O), lambda t: (t, 0)),
        out_shape=jax.ShapeDtypeStruct((B, O), jnp.float32),
    )(crows, g2d)

    return y, loss2d[0, 0]
